# baseline (speedup 1.0000x reference)
import functools

import jax
import jax.numpy as jnp
from jax import lax
from jax.experimental import pallas as pl
from jax.experimental.pallas import tpu as pltpu
from jax.experimental.pallas import tpu_sc as plsc

N = 10000
E = 160000
HD = 256
ED = 128

_BN = 2000
_BE = 2000

_NC, _NS = 2, 16
_NW = _NC * _NS
_NP = 10240
_GC = 40
_GNB = 4
_SC_CH = 80


def _sc_mesh():
    return plsc.VectorSubcoreMesh(core_axis_name="c", subcore_axis_name="s")


def _gather_body(nidx, per_w, nch, *refs):
    h_hbm = refs[0]
    idx_hbms = refs[1:1 + nidx]
    outs = refs[1 + nidx:1 + 2 * nidx]
    sc = refs[1 + 2 * nidx:]
    nb = _GNB
    idx_vs = sc[:nidx]
    bufs = sc[nidx:nidx + nb * nidx]
    gsems = sc[nidx + nb * nidx:nidx + 2 * nb * nidx]
    wsems = sc[nidx + 2 * nb * nidx:nidx + 3 * nb * nidx]

    wid = lax.axis_index("s") * _NC + lax.axis_index("c")
    base = pl.multiple_of(wid * per_w, 8)

    for a in range(nidx):
        pltpu.sync_copy(idx_hbms[a].at[pl.ds(base, per_w)], idx_vs[a])

    def g_start(a, ch, b):
        off = pl.multiple_of(ch * _GC, 8)
        pltpu.async_copy(h_hbm.at[idx_vs[a].at[pl.ds(off, _GC)]],
                         bufs[nb * a + b], gsems[nb * a + b])

    def g_wait(a, b):
        pltpu.make_async_copy(h_hbm.at[idx_vs[a].at[pl.ds(0, _GC)]],
                              bufs[nb * a + b], gsems[nb * a + b]).wait()

    def w_start(a, ch, b):
        pltpu.async_copy(bufs[nb * a + b],
                         outs[a].at[pl.ds(base + ch * _GC, _GC)],
                         wsems[nb * a + b])

    def w_wait(a, b):
        pltpu.make_async_copy(bufs[nb * a + b],
                              outs[a].at[pl.ds(0, _GC)],
                              wsems[nb * a + b]).wait()

    for a in range(nidx):
        for b in range(nb):
            g_start(a, b, b)

    nfull = ((nch - 1) // nb) * nb

    @pl.loop(0, nfull, step=nb)
    def _(ch):
        for b in range(nb):
            c2 = ch + b
            for a in range(nidx):
                g_wait(a, b)
                w_start(a, c2, b)
            for a in range(nidx):
                w_wait(a, b)

                @pl.when(c2 + nb < nch)
                def _():
                    g_start(a, c2 + nb, b)

    for c2 in range(nfull, nch):
        b = c2 % nb
        for a in range(nidx):
            g_wait(a, b)
            w_start(a, c2, b)
        for a in range(nidx):
            w_wait(a, b)


def _sc_gather(h, idxs):
    nidx = len(idxs)
    d = h.shape[1]
    per_w = E // _NW
    nch = per_w // _GC
    scratch = []
    scratch += [pltpu.VMEM((per_w,), jnp.int32) for _ in range(nidx)]
    scratch += [pltpu.VMEM((_GC, d), jnp.float32)
                for _ in range(_GNB * nidx)]
    scratch += [pltpu.SemaphoreType.DMA for _ in range(2 * _GNB * nidx)]
    fn = pl.kernel(
        functools.partial(_gather_body, nidx, per_w, nch),
        out_type=tuple(jax.ShapeDtypeStruct((E, d), jnp.float32)
                       for _ in range(nidx)),
        mesh=_sc_mesh(),
        scratch_types=scratch,
    )
    return fn(h, *idxs)


def _scatter_pipe(sid, msg_hbm, out_hbm, shared, idx_v, mb, lsems, ssem):
    rows0 = pl.multiple_of(sid * (_NP // _NS), 8)
    ebase = sid * (E // _NS)

    for j in range(8):
        pltpu.sync_copy(mb[0], shared.at[pl.ds(rows0 + j * _SC_CH, _SC_CH)])
    plsc.subcore_barrier()

    def l_start(ch, b):
        pltpu.async_copy(msg_hbm.at[pl.ds(ebase + ch * _SC_CH, _SC_CH)],
                         mb[b], lsems[b])

    def l_wait(b):
        pltpu.make_async_copy(msg_hbm.at[pl.ds(0, _SC_CH)], mb[b],
                              lsems[b]).wait()

    nch = (E // _NS) // _SC_CH
    l_start(0, 0)
    l_start(1, 1)

    @pl.loop(0, nch - 1, step=2)
    def _(ch):
        for b in (0, 1):
            c2 = ch + b
            l_wait(b)
            pltpu.async_copy(mb[b], shared.at[idx_v.at[c2]], ssem, add=True)
            pltpu.make_async_copy(mb[b], shared.at[idx_v.at[0]], ssem).wait()

            @pl.when(c2 + 2 < nch)
            def _():
                l_start(c2 + 2, b)

    l_wait(0)
    pltpu.async_copy(mb[0], shared.at[idx_v.at[nch - 1]], ssem, add=True)
    pltpu.make_async_copy(mb[0], shared.at[idx_v.at[0]], ssem).wait()

    plsc.subcore_barrier()

    for j in range(8):
        b = j % 2
        pltpu.sync_copy(shared.at[pl.ds(rows0 + j * _SC_CH, _SC_CH)], mb[b])
        pltpu.sync_copy(mb[b], out_hbm.at[pl.ds(rows0 + j * _SC_CH, _SC_CH)])


def _scatter_body(msg0, msg1, srcr, zeros_hbm, out0, out1,
                  shared, idx_v, mb0, mb1, lsem0, lsem1, ssem):
    cid = lax.axis_index("c")
    sid = lax.axis_index("s")
    pltpu.sync_copy(srcr.at[sid], idx_v)
    pltpu.sync_copy(zeros_hbm, mb0)

    @pl.when(cid == 0)
    def _():
        _scatter_pipe(sid, msg0, out0, shared, idx_v, (mb0, mb1),
                      (lsem0, lsem1), ssem)

    @pl.when(cid == 1)
    def _():
        _scatter_pipe(sid, msg1, out1, shared, idx_v, (mb0, mb1),
                      (lsem0, lsem1), ssem)


def _sc_scatter_add(msg0, msg1, srcr, zeros):
    fn = pl.kernel(
        _scatter_body,
        out_type=(jax.ShapeDtypeStruct((_NP, ED), jnp.float32),
                  jax.ShapeDtypeStruct((_NP, ED), jnp.float32)),
        mesh=_sc_mesh(),
        scratch_types=[
            pltpu.VMEM_SHARED((_NP, ED), jnp.float32),
            pltpu.VMEM((125, _SC_CH), jnp.int32),
            pltpu.VMEM((_SC_CH, ED), jnp.float32),
            pltpu.VMEM((_SC_CH, ED), jnp.float32),
            pltpu.SemaphoreType.DMA,
            pltpu.SemaphoreType.DMA,
            pltpu.SemaphoreType.DMA,
        ],
    )
    return fn(msg0, msg1, srcr, zeros)


def _lrelu(x):
    return jnp.where(x > 0, x, 0.2 * x)


def _dot3(x, w):
    xh = x.astype(jnp.bfloat16)
    xl = (x - xh.astype(jnp.float32)).astype(jnp.bfloat16)
    wh = w.astype(jnp.bfloat16)
    wl = (w - wh.astype(jnp.float32)).astype(jnp.bfloat16)
    return (jnp.dot(xh, wh, preferred_element_type=jnp.float32)
            + jnp.dot(xh, wl, preferred_element_type=jnp.float32)
            + jnp.dot(xl, wh, preferred_element_type=jnp.float32))


def _mlp2_body(n_in, act, ln, residual, nout, *refs):
    xs = refs[:n_in]
    ws = refs[n_in:2 * n_in]
    b1 = refs[2 * n_in]
    w2 = refs[2 * n_in + 1]
    b2 = refs[2 * n_in + 2]
    k = 2 * n_in + 3
    if ln:
        g_ref, bb_ref = refs[k], refs[k + 1]
        k += 2
    if residual:
        res_ref = refs[k]
        k += 1
    out_refs = refs[k:k + nout]

    acc = b1[...].astype(jnp.float32)
    for x_ref, w_ref in zip(xs, ws):
        acc = acc + _dot3(x_ref[...], w_ref[...])
    y = act(acc)
    out = _dot3(y, w2[...]) + b2[...]
    if ln:
        m = jnp.mean(out, axis=-1, keepdims=True)
        v = jnp.mean((out - m) ** 2, axis=-1, keepdims=True)
        out = (out - m) * lax.rsqrt(v + 1e-5) * g_ref[...] + bb_ref[...]
    if residual:
        out = out + res_ref[...]
    if nout == 1:
        out_refs[0][...] = out
    else:
        off = 0
        for o_ref in out_refs:
            w = o_ref.shape[1]
            o_ref[...] = out[:, off:off + w]
            off += w


def _mlp2(xs, w1s, b1, w2, b2, *, act=_lrelu, ln=None, res=None,
          block_rows=_BE, out_split=None):
    rows = xs[0].shape[0]
    assert rows % block_rows == 0
    out_dim = w2.shape[1]
    n_in = len(xs)
    grid = (rows // block_rows,)
    widths = out_split if out_split is not None else (out_dim,)

    in_specs = [pl.BlockSpec((block_rows, x.shape[1]), lambda i: (i, 0))
                for x in xs]
    in_specs += [pl.BlockSpec(w.shape, lambda i: (0, 0)) for w in w1s]
    operands = list(xs) + list(w1s)
    b1r = b1.reshape(1, -1)
    b2r = b2.reshape(1, -1)
    in_specs += [pl.BlockSpec(b1r.shape, lambda i: (0, 0)),
                 pl.BlockSpec(w2.shape, lambda i: (0, 0)),
                 pl.BlockSpec(b2r.shape, lambda i: (0, 0))]
    operands += [b1r, w2, b2r]
    if ln is not None:
        g, bb = ln
        gr, bbr = g.reshape(1, -1), bb.reshape(1, -1)
        in_specs += [pl.BlockSpec(gr.shape, lambda i: (0, 0)),
                     pl.BlockSpec(bbr.shape, lambda i: (0, 0))]
        operands += [gr, bbr]
    if res is not None:
        in_specs.append(pl.BlockSpec((block_rows, out_dim), lambda i: (i, 0)))
        operands.append(res)

    out = pl.pallas_call(
        functools.partial(_mlp2_body, n_in, act, ln is not None,
                          res is not None, len(widths)),
        grid=grid,
        in_specs=in_specs,
        out_specs=[pl.BlockSpec((block_rows, w), lambda i: (i, 0))
                   for w in widths],
        out_shape=[jax.ShapeDtypeStruct((rows, w), jnp.float32)
                   for w in widths],
    )(*operands)
    return out[0] if out_split is None else out


def kernel(node_visuals, edge_index, edge_spatials, params):
    src = edge_index[0]
    dst = edge_index[1]

    ne = params["node_enc"]
    h = _mlp2([node_visuals], [ne["l1"]["W"]], ne["l1"]["b"],
              ne["l2"]["W"], ne["l2"]["b"], ln=(ne["ln_g"], ne["ln_b"]),
              block_rows=_BN)

    ee = params["edge_enc"]
    es_pad = jnp.pad(edge_spatials, ((0, 0), (0, 5)))
    w1_pad = jnp.pad(ee["l1"]["W"], ((0, 5), (0, 0)))
    e = _mlp2([es_pad], [w1_pad], ee["l1"]["b"],
              ee["l2"]["W"], ee["l2"]["b"], ln=(ee["ln_g"], ee["ln_b"]))

    srcr = src.reshape(_NS, (E // _NS) // _SC_CH, _SC_CH)
    zeros = jnp.zeros((_SC_CH, ED), jnp.float32)

    (hd,) = _sc_gather(h, [dst])
    for lp in params["layers"]:
        msg0, msg1 = _mlp2([hd, e],
                           [lp["msg1"]["W"][:HD], lp["msg1"]["W"][HD:]],
                           lp["msg1"]["b"], lp["msg2"]["W"], lp["msg2"]["b"],
                           out_split=(ED, ED))
        agg0, agg1 = _sc_scatter_add(msg0, msg1, srcr, zeros)
        agg0, agg1 = agg0[:N], agg1[:N]
        uw = lp["upd1"]["W"]
        h = _mlp2([h, agg0, agg1],
                  [uw[:HD], uw[HD:HD + ED], uw[HD + ED:]],
                  lp["upd1"]["b"], lp["upd2"]["W"], lp["upd2"]["b"],
                  res=h, block_rows=_BN)
        hs, hd = _sc_gather(h, [src, dst])
        e = _mlp2([hs, hd, e],
                  [lp["e1"]["W"][:HD], lp["e1"]["W"][HD:2 * HD],
                   lp["e1"]["W"][2 * HD:]],
                  lp["e1"]["b"], lp["e2"]["W"], lp["e2"]["b"], res=e)

    c1 = params["cls1"]
    probs = _mlp2([hs, hd, e],
                  [c1["W"][:HD], c1["W"][HD:2 * HD], c1["W"][2 * HD:]],
                  c1["b"], params["cls2"]["W"], params["cls2"]["b"],
                  act=lambda x: jnp.maximum(x, 0.0))
    return probs


# speedup vs baseline: 1.5013x; 1.5013x over previous
import functools

import jax
import jax.numpy as jnp
from jax import lax
from jax.experimental import pallas as pl
from jax.experimental.pallas import tpu as pltpu
from jax.experimental.pallas import tpu_sc as plsc

N = 10000
E = 160000
HD = 256
ED = 128

_BN = 2000
_BE = 2000

_NC, _NS = 2, 16
_NW = _NC * _NS
_NP = 10240
_GC = 40
_GNB = 4
_SC_CH = 80


def _sc_mesh():
    return plsc.VectorSubcoreMesh(core_axis_name="c", subcore_axis_name="s")


def _gather_body(nidx, per_w, nch, *refs):
    h_hbm = refs[0]
    idx_hbms = refs[1:1 + nidx]
    outs = refs[1 + nidx:1 + 2 * nidx]
    sc = refs[1 + 2 * nidx:]
    nb = _GNB
    idx_vs = sc[:nidx]
    bufs = sc[nidx:nidx + nb * nidx]
    gsems = sc[nidx + nb * nidx:nidx + 2 * nb * nidx]
    wsems = sc[nidx + 2 * nb * nidx:nidx + 3 * nb * nidx]

    wid = lax.axis_index("s") * _NC + lax.axis_index("c")
    base = pl.multiple_of(wid * per_w, 8)

    for a in range(nidx):
        pltpu.sync_copy(idx_hbms[a].at[pl.ds(base, per_w)], idx_vs[a])

    def g_start(a, ch, b):
        off = pl.multiple_of(ch * _GC, 8)
        pltpu.async_copy(h_hbm.at[idx_vs[a].at[pl.ds(off, _GC)]],
                         bufs[nb * a + b], gsems[nb * a + b])

    def g_wait(a, b):
        pltpu.make_async_copy(h_hbm.at[idx_vs[a].at[pl.ds(0, _GC)]],
                              bufs[nb * a + b], gsems[nb * a + b]).wait()

    def w_start(a, ch, b):
        pltpu.async_copy(bufs[nb * a + b],
                         outs[a].at[pl.ds(base + ch * _GC, _GC)],
                         wsems[nb * a + b])

    def w_wait(a, b):
        pltpu.make_async_copy(bufs[nb * a + b],
                              outs[a].at[pl.ds(0, _GC)],
                              wsems[nb * a + b]).wait()

    lead = nb // 2

    def step(c2, b, static):
        bg = (b + lead) % nb
        for a in range(nidx):
            if static:
                if c2 + lead < nch:
                    if c2 + lead - nb >= 0:
                        w_wait(a, bg)
                    g_start(a, c2 + lead, bg)
            else:
                @pl.when(c2 + lead < nch)
                def _():
                    @pl.when(c2 + lead - nb >= 0)
                    def _():
                        w_wait(a, bg)
                    g_start(a, c2 + lead, bg)
        for a in range(nidx):
            g_wait(a, b)
            w_start(a, c2, b)

    for a in range(nidx):
        for b in range(lead):
            g_start(a, b, b)

    nfull = (nch // nb) * nb

    @pl.loop(0, nfull, step=nb)
    def _(ch0):
        for b in range(nb):
            step(ch0 + b, b, static=False)

    for c2 in range(nfull, nch):
        step(c2, c2 % nb, static=True)

    for c2 in range(max(0, nch - nb), nch):
        for a in range(nidx):
            w_wait(a, c2 % nb)


def _sc_gather(h, idxs):
    nidx = len(idxs)
    d = h.shape[1]
    per_w = E // _NW
    nch = per_w // _GC
    scratch = []
    scratch += [pltpu.VMEM((per_w,), jnp.int32) for _ in range(nidx)]
    scratch += [pltpu.VMEM((_GC, d), jnp.float32)
                for _ in range(_GNB * nidx)]
    scratch += [pltpu.SemaphoreType.DMA for _ in range(2 * _GNB * nidx)]
    fn = pl.kernel(
        functools.partial(_gather_body, nidx, per_w, nch),
        out_type=tuple(jax.ShapeDtypeStruct((E, d), jnp.float32)
                       for _ in range(nidx)),
        mesh=_sc_mesh(),
        scratch_types=scratch,
    )
    return fn(h, *idxs)


def _scatter_pipe(sid, msg_hbm, out_hbm, shared, idx_v, mb, lsems, ssems):
    rows0 = pl.multiple_of(sid * (_NP // _NS), 8)
    ebase = sid * (E // _NS)

    for j in range(8):
        pltpu.sync_copy(mb[0], shared.at[pl.ds(rows0 + j * _SC_CH, _SC_CH)])
    plsc.subcore_barrier()

    nb = len(mb)
    lead = 1

    def l_start(ch, b):
        pltpu.async_copy(msg_hbm.at[pl.ds(ebase + ch * _SC_CH, _SC_CH)],
                         mb[b], lsems[b])

    def l_wait(b):
        pltpu.make_async_copy(msg_hbm.at[pl.ds(0, _SC_CH)], mb[b],
                              lsems[b]).wait()

    def s_start(ch, b):
        pltpu.async_copy(mb[b], shared.at[idx_v.at[ch]], ssems[b], add=True)

    def s_wait(b):
        pltpu.make_async_copy(mb[b], shared.at[idx_v.at[0]],
                              ssems[b]).wait()

    nch = (E // _NS) // _SC_CH

    def step(c2, b, static):
        bg = (b + lead) % nb
        if static:
            if c2 + lead < nch:
                if c2 + lead - nb >= 0:
                    s_wait(bg)
                l_start(c2 + lead, bg)
        else:
            @pl.when(c2 + lead < nch)
            def _():
                @pl.when(c2 + lead - nb >= 0)
                def _():
                    s_wait(bg)
                l_start(c2 + lead, bg)
        l_wait(b)
        s_start(c2, b)

    for b in range(lead):
        l_start(b, b)

    nfull = (nch // nb) * nb

    @pl.loop(0, nfull, step=nb)
    def _(ch0):
        for b in range(nb):
            step(ch0 + b, b, static=False)

    for c2 in range(nfull, nch):
        step(c2, c2 % nb, static=True)

    for c2 in range(max(0, nch - nb), nch):
        s_wait(c2 % nb)

    plsc.subcore_barrier()

    for j in range(8):
        b = j % 2
        pltpu.sync_copy(shared.at[pl.ds(rows0 + j * _SC_CH, _SC_CH)], mb[b])
        pltpu.sync_copy(mb[b], out_hbm.at[pl.ds(rows0 + j * _SC_CH, _SC_CH)])


def _scatter_body(msg0, msg1, srcr, zeros_hbm, out0, out1,
                  shared, idx_v, mb0, mb1, mb2,
                  lsem0, lsem1, lsem2, ssem0, ssem1, ssem2):
    cid = lax.axis_index("c")
    sid = lax.axis_index("s")
    pltpu.sync_copy(srcr.at[sid], idx_v)
    pltpu.sync_copy(zeros_hbm, mb0)

    @pl.when(cid == 0)
    def _():
        _scatter_pipe(sid, msg0, out0, shared, idx_v, (mb0, mb1, mb2),
                      (lsem0, lsem1, lsem2), (ssem0, ssem1, ssem2))

    @pl.when(cid == 1)
    def _():
        _scatter_pipe(sid, msg1, out1, shared, idx_v, (mb0, mb1, mb2),
                      (lsem0, lsem1, lsem2), (ssem0, ssem1, ssem2))


def _sc_scatter_add(msg0, msg1, srcr, zeros):
    fn = pl.kernel(
        _scatter_body,
        out_type=(jax.ShapeDtypeStruct((_NP, ED), jnp.float32),
                  jax.ShapeDtypeStruct((_NP, ED), jnp.float32)),
        mesh=_sc_mesh(),
        scratch_types=[
            pltpu.VMEM_SHARED((_NP, ED), jnp.float32),
            pltpu.VMEM((125, _SC_CH), jnp.int32),
            pltpu.VMEM((_SC_CH, ED), jnp.float32),
            pltpu.VMEM((_SC_CH, ED), jnp.float32),
            pltpu.VMEM((_SC_CH, ED), jnp.float32),
            pltpu.SemaphoreType.DMA,
            pltpu.SemaphoreType.DMA,
            pltpu.SemaphoreType.DMA,
            pltpu.SemaphoreType.DMA,
            pltpu.SemaphoreType.DMA,
            pltpu.SemaphoreType.DMA,
        ],
    )
    return fn(msg0, msg1, srcr, zeros)


def _lrelu(x):
    return jnp.where(x > 0, x, 0.2 * x)


def _mlp2_body(n_in, act, ln, residual, nout, *refs):
    xs = refs[:n_in]
    ws = refs[n_in:2 * n_in]
    b1 = refs[2 * n_in]
    w2 = refs[2 * n_in + 1]
    b2 = refs[2 * n_in + 2]
    k = 2 * n_in + 3
    if ln:
        g_ref, bb_ref = refs[k], refs[k + 1]
        k += 2
    if residual:
        res_ref = refs[k]
        k += 1
    out_refs = refs[k:k + nout]

    acc = b1[...].astype(jnp.float32)
    for x_ref, w_ref in zip(xs, ws):
        acc = acc + jnp.dot(x_ref[...], w_ref[...],
                            preferred_element_type=jnp.float32)
    y = act(acc)
    out = jnp.dot(y, w2[...], preferred_element_type=jnp.float32) + b2[...]
    if ln:
        m = jnp.mean(out, axis=-1, keepdims=True)
        v = jnp.mean((out - m) ** 2, axis=-1, keepdims=True)
        out = (out - m) * lax.rsqrt(v + 1e-5) * g_ref[...] + bb_ref[...]
    if residual:
        out = out + res_ref[...]
    if nout == 1:
        out_refs[0][...] = out
    else:
        off = 0
        for o_ref in out_refs:
            w = o_ref.shape[1]
            o_ref[...] = out[:, off:off + w]
            off += w


def _mlp2(xs, w1s, b1, w2, b2, *, act=_lrelu, ln=None, res=None,
          block_rows=_BE, out_split=None):
    rows = xs[0].shape[0]
    assert rows % block_rows == 0
    out_dim = w2.shape[1]
    n_in = len(xs)
    grid = (rows // block_rows,)
    widths = out_split if out_split is not None else (out_dim,)

    in_specs = [pl.BlockSpec((block_rows, x.shape[1]), lambda i: (i, 0))
                for x in xs]
    in_specs += [pl.BlockSpec(w.shape, lambda i: (0, 0)) for w in w1s]
    operands = list(xs) + list(w1s)
    b1r = b1.reshape(1, -1)
    b2r = b2.reshape(1, -1)
    in_specs += [pl.BlockSpec(b1r.shape, lambda i: (0, 0)),
                 pl.BlockSpec(w2.shape, lambda i: (0, 0)),
                 pl.BlockSpec(b2r.shape, lambda i: (0, 0))]
    operands += [b1r, w2, b2r]
    if ln is not None:
        g, bb = ln
        gr, bbr = g.reshape(1, -1), bb.reshape(1, -1)
        in_specs += [pl.BlockSpec(gr.shape, lambda i: (0, 0)),
                     pl.BlockSpec(bbr.shape, lambda i: (0, 0))]
        operands += [gr, bbr]
    if res is not None:
        in_specs.append(pl.BlockSpec((block_rows, out_dim), lambda i: (i, 0)))
        operands.append(res)

    out = pl.pallas_call(
        functools.partial(_mlp2_body, n_in, act, ln is not None,
                          res is not None, len(widths)),
        grid=grid,
        in_specs=in_specs,
        out_specs=[pl.BlockSpec((block_rows, w), lambda i: (i, 0))
                   for w in widths],
        out_shape=[jax.ShapeDtypeStruct((rows, w), jnp.float32)
                   for w in widths],
    )(*operands)
    return out[0] if out_split is None else out


def kernel(node_visuals, edge_index, edge_spatials, params):
    src = edge_index[0]
    dst = edge_index[1]

    ne = params["node_enc"]
    h = _mlp2([node_visuals], [ne["l1"]["W"]], ne["l1"]["b"],
              ne["l2"]["W"], ne["l2"]["b"], ln=(ne["ln_g"], ne["ln_b"]),
              block_rows=_BN)

    ee = params["edge_enc"]
    es_pad = jnp.pad(edge_spatials, ((0, 0), (0, 5)))
    w1_pad = jnp.pad(ee["l1"]["W"], ((0, 5), (0, 0)))
    e = _mlp2([es_pad], [w1_pad], ee["l1"]["b"],
              ee["l2"]["W"], ee["l2"]["b"], ln=(ee["ln_g"], ee["ln_b"]))

    srcr = src.reshape(_NS, (E // _NS) // _SC_CH, _SC_CH)
    zeros = jnp.zeros((_SC_CH, ED), jnp.float32)

    (hd,) = _sc_gather(h, [dst])
    for lp in params["layers"]:
        msg0, msg1 = _mlp2([hd, e],
                           [lp["msg1"]["W"][:HD], lp["msg1"]["W"][HD:]],
                           lp["msg1"]["b"], lp["msg2"]["W"], lp["msg2"]["b"],
                           out_split=(ED, ED))
        agg0, agg1 = _sc_scatter_add(msg0, msg1, srcr, zeros)
        agg0, agg1 = agg0[:N], agg1[:N]
        uw = lp["upd1"]["W"]
        h = _mlp2([h, agg0, agg1],
                  [uw[:HD], uw[HD:HD + ED], uw[HD + ED:]],
                  lp["upd1"]["b"], lp["upd2"]["W"], lp["upd2"]["b"],
                  res=h, block_rows=_BN)
        hs, hd = _sc_gather(h, [src, dst])
        e = _mlp2([hs, hd, e],
                  [lp["e1"]["W"][:HD], lp["e1"]["W"][HD:2 * HD],
                   lp["e1"]["W"][2 * HD:]],
                  lp["e1"]["b"], lp["e2"]["W"], lp["e2"]["b"], res=e)

    c1 = params["cls1"]
    probs = _mlp2([hs, hd, e],
                  [c1["W"][:HD], c1["W"][HD:2 * HD], c1["W"][2 * HD:]],
                  c1["b"], params["cls2"]["W"], params["cls2"]["b"],
                  act=lambda x: jnp.maximum(x, 0.0))
    return probs


# speedup vs baseline: 1.8015x; 1.1999x over previous
import functools

import jax
import jax.numpy as jnp
from jax import lax
from jax.experimental import pallas as pl
from jax.experimental.pallas import tpu as pltpu
from jax.experimental.pallas import tpu_sc as plsc

N = 10000
E = 160000
HD = 256
ED = 128

_BN = 2000
_BE = 2000

_NC, _NS = 2, 16
_NW = _NC * _NS
_NP = 10240
_GC = 40
_GNB = 4
_SC_CH = 80


def _sc_mesh():
    return plsc.VectorSubcoreMesh(core_axis_name="c", subcore_axis_name="s")


def _gather_body(nidx, per_w, nch, *refs):
    h_hbm = refs[0]
    idx_hbms = refs[1:1 + nidx]
    outs = refs[1 + nidx:1 + 2 * nidx]
    sc = refs[1 + 2 * nidx:]
    nb = _GNB
    idx_vs = sc[:nidx]
    bufs = sc[nidx:nidx + nb * nidx]
    gsems = sc[nidx + nb * nidx:nidx + 2 * nb * nidx]
    wsems = sc[nidx + 2 * nb * nidx:nidx + 3 * nb * nidx]

    wid = lax.axis_index("s") * _NC + lax.axis_index("c")
    base = pl.multiple_of(wid * per_w, 8)

    for a in range(nidx):
        pltpu.sync_copy(idx_hbms[a].at[pl.ds(base, per_w)], idx_vs[a])

    def g_start(a, ch, b):
        off = pl.multiple_of(ch * _GC, 8)
        pltpu.async_copy(h_hbm.at[idx_vs[a].at[pl.ds(off, _GC)]],
                         bufs[nb * a + b], gsems[nb * a + b])

    def g_wait(a, b):
        pltpu.make_async_copy(h_hbm.at[idx_vs[a].at[pl.ds(0, _GC)]],
                              bufs[nb * a + b], gsems[nb * a + b]).wait()

    def w_start(a, ch, b):
        pltpu.async_copy(bufs[nb * a + b],
                         outs[a].at[pl.ds(base + ch * _GC, _GC)],
                         wsems[nb * a + b])

    def w_wait(a, b):
        pltpu.make_async_copy(bufs[nb * a + b],
                              outs[a].at[pl.ds(0, _GC)],
                              wsems[nb * a + b]).wait()

    lead = nb // 2

    def step(c2, b, static):
        bg = (b + lead) % nb
        for a in range(nidx):
            if static:
                if c2 + lead < nch:
                    if c2 + lead - nb >= 0:
                        w_wait(a, bg)
                    g_start(a, c2 + lead, bg)
            else:
                @pl.when(c2 + lead < nch)
                def _():
                    @pl.when(c2 + lead - nb >= 0)
                    def _():
                        w_wait(a, bg)
                    g_start(a, c2 + lead, bg)
        for a in range(nidx):
            g_wait(a, b)
            w_start(a, c2, b)

    for a in range(nidx):
        for b in range(lead):
            g_start(a, b, b)

    nfull = (nch // nb) * nb

    @pl.loop(0, nfull, step=nb)
    def _(ch0):
        for b in range(nb):
            step(ch0 + b, b, static=False)

    for c2 in range(nfull, nch):
        step(c2, c2 % nb, static=True)

    for c2 in range(max(0, nch - nb), nch):
        for a in range(nidx):
            w_wait(a, c2 % nb)


def _sc_gather(h, idxs):
    nidx = len(idxs)
    d = h.shape[1]
    per_w = E // _NW
    nch = per_w // _GC
    scratch = []
    scratch += [pltpu.VMEM((per_w,), jnp.int32) for _ in range(nidx)]
    scratch += [pltpu.VMEM((_GC, d), jnp.float32)
                for _ in range(_GNB * nidx)]
    scratch += [pltpu.SemaphoreType.DMA for _ in range(2 * _GNB * nidx)]
    fn = pl.kernel(
        functools.partial(_gather_body, nidx, per_w, nch),
        out_type=tuple(jax.ShapeDtypeStruct((E, d), jnp.float32)
                       for _ in range(nidx)),
        mesh=_sc_mesh(),
        scratch_types=scratch,
    )
    return fn(h, *idxs)


def _scatter_pipe(sid, msg_hbm, out_hbm, shared, idx_v, mb, lsems, ssems):
    rows0 = pl.multiple_of(sid * (_NP // _NS), 8)
    ebase = sid * (E // _NS)

    for j in range(8):
        pltpu.sync_copy(mb[0], shared.at[pl.ds(rows0 + j * _SC_CH, _SC_CH)])
    plsc.subcore_barrier()

    nb = len(mb)
    lead = 1

    def l_start(ch, b):
        pltpu.async_copy(msg_hbm.at[pl.ds(ebase + ch * _SC_CH, _SC_CH)],
                         mb[b], lsems[b])

    def l_wait(b):
        pltpu.make_async_copy(msg_hbm.at[pl.ds(0, _SC_CH)], mb[b],
                              lsems[b]).wait()

    def s_start(ch, b):
        pltpu.async_copy(mb[b], shared.at[idx_v.at[ch]], ssems[b], add=True)

    def s_wait(b):
        pltpu.make_async_copy(mb[b], shared.at[idx_v.at[0]],
                              ssems[b]).wait()

    nch = (E // _NS) // _SC_CH

    def step(c2, b, static):
        bg = (b + lead) % nb
        if static:
            if c2 + lead < nch:
                if c2 + lead - nb >= 0:
                    s_wait(bg)
                l_start(c2 + lead, bg)
        else:
            @pl.when(c2 + lead < nch)
            def _():
                @pl.when(c2 + lead - nb >= 0)
                def _():
                    s_wait(bg)
                l_start(c2 + lead, bg)
        l_wait(b)
        s_start(c2, b)

    for b in range(lead):
        l_start(b, b)

    nfull = (nch // nb) * nb

    @pl.loop(0, nfull, step=nb)
    def _(ch0):
        for b in range(nb):
            step(ch0 + b, b, static=False)

    for c2 in range(nfull, nch):
        step(c2, c2 % nb, static=True)

    for c2 in range(max(0, nch - nb), nch):
        s_wait(c2 % nb)

    plsc.subcore_barrier()

    for j in range(8):
        b = j % 2
        pltpu.sync_copy(shared.at[pl.ds(rows0 + j * _SC_CH, _SC_CH)], mb[b])
        pltpu.sync_copy(mb[b], out_hbm.at[pl.ds(rows0 + j * _SC_CH, _SC_CH)])


def _scatter_body(msg0, msg1, srcr, zeros_hbm, out0, out1,
                  shared, idx_v, mb0, mb1, mb2,
                  lsem0, lsem1, lsem2, ssem0, ssem1, ssem2):
    cid = lax.axis_index("c")
    sid = lax.axis_index("s")
    pltpu.sync_copy(srcr.at[sid], idx_v)
    pltpu.sync_copy(zeros_hbm, mb0)

    @pl.when(cid == 0)
    def _():
        _scatter_pipe(sid, msg0, out0, shared, idx_v, (mb0, mb1, mb2),
                      (lsem0, lsem1, lsem2), (ssem0, ssem1, ssem2))

    @pl.when(cid == 1)
    def _():
        _scatter_pipe(sid, msg1, out1, shared, idx_v, (mb0, mb1, mb2),
                      (lsem0, lsem1, lsem2), (ssem0, ssem1, ssem2))


def _sc_scatter_add(msg0, msg1, srcr, zeros):
    fn = pl.kernel(
        _scatter_body,
        out_type=(jax.ShapeDtypeStruct((_NP, ED), jnp.float32),
                  jax.ShapeDtypeStruct((_NP, ED), jnp.float32)),
        mesh=_sc_mesh(),
        scratch_types=[
            pltpu.VMEM_SHARED((_NP, ED), jnp.float32),
            pltpu.VMEM((125, _SC_CH), jnp.int32),
            pltpu.VMEM((_SC_CH, ED), jnp.float32),
            pltpu.VMEM((_SC_CH, ED), jnp.float32),
            pltpu.VMEM((_SC_CH, ED), jnp.float32),
            pltpu.SemaphoreType.DMA,
            pltpu.SemaphoreType.DMA,
            pltpu.SemaphoreType.DMA,
            pltpu.SemaphoreType.DMA,
            pltpu.SemaphoreType.DMA,
            pltpu.SemaphoreType.DMA,
        ],
    )
    return fn(msg0, msg1, srcr, zeros)


def _lrelu(x):
    return jnp.where(x > 0, x, 0.2 * x)


def _mlp2_body(n_in, act, ln, residual, nout, *refs):
    xs = refs[:n_in]
    ws = refs[n_in:2 * n_in]
    b1 = refs[2 * n_in]
    w2 = refs[2 * n_in + 1]
    b2 = refs[2 * n_in + 2]
    k = 2 * n_in + 3
    if ln:
        g_ref, bb_ref = refs[k], refs[k + 1]
        k += 2
    if residual:
        res_ref = refs[k]
        k += 1
    out_refs = refs[k:k + nout]

    acc = b1[...].astype(jnp.float32)
    for x_ref, w_ref in zip(xs, ws):
        acc = acc + jnp.dot(x_ref[...], w_ref[...],
                            preferred_element_type=jnp.float32)
    y = act(acc)
    out = jnp.dot(y, w2[...], preferred_element_type=jnp.float32) + b2[...]
    if ln:
        m = jnp.mean(out, axis=-1, keepdims=True)
        v = jnp.mean((out - m) ** 2, axis=-1, keepdims=True)
        out = (out - m) * lax.rsqrt(v + 1e-5) * g_ref[...] + bb_ref[...]
    if residual:
        out = out + res_ref[...]
    if nout == 1:
        out_refs[0][...] = out
    else:
        off = 0
        for o_ref in out_refs:
            w = o_ref.shape[1]
            o_ref[...] = out[:, off:off + w]
            off += w


def _mlp2(xs, w1s, b1, w2, b2, *, act=_lrelu, ln=None, res=None,
          block_rows=_BE, out_split=None):
    rows = xs[0].shape[0]
    assert rows % block_rows == 0
    out_dim = w2.shape[1]
    n_in = len(xs)
    grid = (rows // block_rows,)
    widths = out_split if out_split is not None else (out_dim,)

    in_specs = [pl.BlockSpec((block_rows, x.shape[1]), lambda i: (i, 0))
                for x in xs]
    in_specs += [pl.BlockSpec(w.shape, lambda i: (0, 0)) for w in w1s]
    operands = list(xs) + list(w1s)
    b1r = b1.reshape(1, -1)
    b2r = b2.reshape(1, -1)
    in_specs += [pl.BlockSpec(b1r.shape, lambda i: (0, 0)),
                 pl.BlockSpec(w2.shape, lambda i: (0, 0)),
                 pl.BlockSpec(b2r.shape, lambda i: (0, 0))]
    operands += [b1r, w2, b2r]
    if ln is not None:
        g, bb = ln
        gr, bbr = g.reshape(1, -1), bb.reshape(1, -1)
        in_specs += [pl.BlockSpec(gr.shape, lambda i: (0, 0)),
                     pl.BlockSpec(bbr.shape, lambda i: (0, 0))]
        operands += [gr, bbr]
    if res is not None:
        in_specs.append(pl.BlockSpec((block_rows, out_dim), lambda i: (i, 0)))
        operands.append(res)

    out = pl.pallas_call(
        functools.partial(_mlp2_body, n_in, act, ln is not None,
                          res is not None, len(widths)),
        grid=grid,
        in_specs=in_specs,
        out_specs=[pl.BlockSpec((block_rows, w), lambda i: (i, 0))
                   for w in widths],
        out_shape=[jax.ShapeDtypeStruct((rows, w), jnp.float32)
                   for w in widths],
    )(*operands)
    return out[0] if out_split is None else out


def _ln(x, g, b):
    m = jnp.mean(x, axis=-1, keepdims=True)
    v = jnp.mean((x - m) ** 2, axis=-1, keepdims=True)
    return (x - m) * lax.rsqrt(v + 1e-5) * g + b


def _mm(xs, ws, b1, w2, b2, act):
    acc = b1
    for x, w in zip(xs, ws):
        acc = acc + jnp.dot(x, w, preferred_element_type=jnp.float32)
    y = act(acc)
    return jnp.dot(y, w2, preferred_element_type=jnp.float32) + b2


def _wspecs(arrs):
    return [pl.BlockSpec(a.shape, lambda i, nd=a.ndim: (0,) * nd)
            for a in arrs]


def _enc_msg_body(refs_in, refs_out):
    (es, hd, w1, b1, w2, b2, g, bb, m1a, m1b, mb1, m2, mb2) = refs_in
    e_out, m0_out, m1_out = refs_out
    e = _ln(_mm([es[...]], [w1[...]], b1[...], w2[...], b2[...], _lrelu),
            g[...], bb[...])
    m = _mm([hd[...], e], [m1a[...], m1b[...]], mb1[...], m2[...], mb2[...],
            _lrelu)
    e_out[...] = e
    m0_out[...] = m[:, :ED]
    m1_out[...] = m[:, ED:]


def _eupd_msg_body(refs_in, refs_out):
    (hs, hd, e, ea, eb, ec, be1, e2, be2, m1a, m1b, mb1, m2, mb2) = refs_in
    e_out, m0_out, m1_out = refs_out
    en = _mm([hs[...], hd[...], e[...]], [ea[...], eb[...], ec[...]],
             be1[...], e2[...], be2[...], _lrelu) + e[...]
    m = _mm([hd[...], en], [m1a[...], m1b[...]], mb1[...], m2[...],
            mb2[...], _lrelu)
    e_out[...] = en
    m0_out[...] = m[:, :ED]
    m1_out[...] = m[:, ED:]


def _eupd_cls_body(refs_in, refs_out):
    (hs, hd, e, ea, eb, ec, be1, e2, be2,
     c1a, c1b, c1c, cb1, c2, cb2) = refs_in
    (p_out,) = refs_out
    en = _mm([hs[...], hd[...], e[...]], [ea[...], eb[...], ec[...]],
             be1[...], e2[...], be2[...], _lrelu) + e[...]
    p = _mm([hs[...], hd[...], en], [c1a[...], c1b[...], c1c[...]],
            cb1[...], c2[...], cb2[...],
            lambda x: jnp.maximum(x, 0.0))
    p_out[...] = p


def _edge_call(body, xs, weights, out_widths):
    n_x = len(xs)

    def wrapped(*refs):
        body(refs[:n_x + len(weights)], refs[n_x + len(weights):])

    in_specs = [pl.BlockSpec((_BE, x.shape[1]), lambda i: (i, 0))
                for x in xs] + _wspecs(weights)
    return pl.pallas_call(
        wrapped,
        grid=(E // _BE,),
        in_specs=in_specs,
        out_specs=[pl.BlockSpec((_BE, w), lambda i: (i, 0))
                   for w in out_widths],
        out_shape=[jax.ShapeDtypeStruct((E, w), jnp.float32)
                   for w in out_widths],
    )(*xs, *weights)


def _r2(b):
    return b.reshape(1, -1)


def kernel(node_visuals, edge_index, edge_spatials, params):
    src = edge_index[0]
    dst = edge_index[1]

    ne = params["node_enc"]
    h = _mlp2([node_visuals], [ne["l1"]["W"]], ne["l1"]["b"],
              ne["l2"]["W"], ne["l2"]["b"], ln=(ne["ln_g"], ne["ln_b"]),
              block_rows=_BN)

    ee = params["edge_enc"]
    es_pad = jnp.pad(edge_spatials, ((0, 0), (0, 5)))
    w1_pad = jnp.pad(ee["l1"]["W"], ((0, 5), (0, 0)))

    srcr = src.reshape(_NS, (E // _NS) // _SC_CH, _SC_CH)
    zeros = jnp.zeros((_SC_CH, ED), jnp.float32)

    layers = params["layers"]

    def msg_w(lp):
        return [lp["msg1"]["W"][:HD], lp["msg1"]["W"][HD:],
                _r2(lp["msg1"]["b"]), lp["msg2"]["W"], _r2(lp["msg2"]["b"])]

    def eupd_w(lp):
        return [lp["e1"]["W"][:HD], lp["e1"]["W"][HD:2 * HD],
                lp["e1"]["W"][2 * HD:], _r2(lp["e1"]["b"]),
                lp["e2"]["W"], _r2(lp["e2"]["b"])]

    (hd,) = _sc_gather(h, [dst])
    e, msg0, msg1 = _edge_call(
        _enc_msg_body, [es_pad, hd],
        [w1_pad, _r2(ee["l1"]["b"]), ee["l2"]["W"], _r2(ee["l2"]["b"]),
         _r2(ee["ln_g"]), _r2(ee["ln_b"])] + msg_w(layers[0]),
        (ED, ED, ED))

    for li, lp in enumerate(layers):
        agg0, agg1 = _sc_scatter_add(msg0, msg1, srcr, zeros)
        agg0, agg1 = agg0[:N], agg1[:N]
        uw = lp["upd1"]["W"]
        h = _mlp2([h, agg0, agg1],
                  [uw[:HD], uw[HD:HD + ED], uw[HD + ED:]],
                  lp["upd1"]["b"], lp["upd2"]["W"], lp["upd2"]["b"],
                  res=h, block_rows=_BN)
        hs, hd = _sc_gather(h, [src, dst])
        if li < len(layers) - 1:
            e, msg0, msg1 = _edge_call(
                _eupd_msg_body, [hs, hd, e],
                eupd_w(lp) + msg_w(layers[li + 1]), (ED, ED, ED))
        else:
            c1 = params["cls1"]
            (probs,) = _edge_call(
                _eupd_cls_body, [hs, hd, e],
                eupd_w(lp) + [c1["W"][:HD], c1["W"][HD:2 * HD],
                              c1["W"][2 * HD:], _r2(c1["b"]),
                              params["cls2"]["W"],
                              _r2(params["cls2"]["b"])],
                (1,))
    return probs
